# speedup vs baseline: 1.6408x; 1.6408x over previous
import jax
import jax.numpy as jnp
from jax.experimental import pallas as pl
from jax.experimental.pallas import tpu as pltpu

B, S, H = 1, 2048, 1024
E = 8
TOP_K = 2
D_FF = 1024
N_SHARED = 2
D_FF_SHARED = D_FF * N_SHARED

TM = 1024
_DN = (((1,), (1,)), ((), ()))


def _routed_kernel(x_ref, gate_ref, wg_ref, wu_ref, wd_ref, out_ref, fw_ref):
    e = pl.program_id(1)
    x = x_ref[...]

    @pl.when(e == 0)
    def _():
        logits = jax.lax.dot_general(x, gate_ref[...], _DN,
                                     preferred_element_type=jnp.float32)
        mx = jnp.max(logits, axis=1, keepdims=True)
        p = jnp.exp(logits - mx)
        p = p / jnp.sum(p, axis=1, keepdims=True)
        cols = jax.lax.broadcasted_iota(jnp.int32, p.shape, 1)
        i1 = jnp.argmax(p, axis=1)
        m1 = jnp.max(p, axis=1)
        oh1 = cols == i1[:, None]
        p2 = jnp.where(oh1, -1.0, p)
        i2 = jnp.argmax(p2, axis=1)
        m2 = jnp.max(p2, axis=1)
        oh2 = cols == i2[:, None]
        denom = m1 + m2 + 1e-20
        fw = (jnp.where(oh1, (m1 / denom)[:, None], 0.0)
              + jnp.where(oh2, (m2 / denom)[:, None], 0.0))
        fw_ref[...] = fw.astype(jnp.float32)
        out_ref[...] = jnp.zeros_like(out_ref)

    g = jax.lax.dot_general(x, wg_ref[0], _DN,
                            preferred_element_type=jnp.float32)
    u = jax.lax.dot_general(x, wu_ref[0], _DN,
                            preferred_element_type=jnp.float32)
    h = (g * jax.nn.sigmoid(g)) * u
    y = jax.lax.dot_general(h, wd_ref[0], _DN,
                            preferred_element_type=jnp.float32)
    cols = jax.lax.broadcasted_iota(jnp.int32, fw_ref.shape, 1)
    w = jnp.sum(jnp.where(cols == e, fw_ref[...], 0.0), axis=1)
    out_ref[...] += y * w[:, None]


def _shared_kernel(x_ref, wsg_ref, wsu_ref, wsd_ref, rout_ref, out_ref):
    c = pl.program_id(1)
    x = x_ref[...]
    g = jax.lax.dot_general(x, wsg_ref[...], _DN,
                            preferred_element_type=jnp.float32)
    u = jax.lax.dot_general(x, wsu_ref[...], _DN,
                            preferred_element_type=jnp.float32)
    h = (g * jax.nn.sigmoid(g)) * u
    y = jax.lax.dot_general(h, wsd_ref[...], _DN,
                            preferred_element_type=jnp.float32)

    @pl.when(c == 0)
    def _():
        out_ref[...] = rout_ref[...] + y

    @pl.when(c == 1)
    def _():
        out_ref[...] += y


def kernel(hidden_states, gate_weight, Wg, Wu, Wd, Wsg, Wsu, Wsd):
    bsz, seq_len, h = hidden_states.shape
    x = hidden_states.reshape(-1, h)
    T = x.shape[0]

    routed = pl.pallas_call(
        _routed_kernel,
        grid=(T // TM, E),
        in_specs=[
            pl.BlockSpec((TM, H), lambda t, e: (t, 0)),
            pl.BlockSpec((E, H), lambda t, e: (0, 0)),
            pl.BlockSpec((1, D_FF, H), lambda t, e: (e, 0, 0)),
            pl.BlockSpec((1, D_FF, H), lambda t, e: (e, 0, 0)),
            pl.BlockSpec((1, H, D_FF), lambda t, e: (e, 0, 0)),
        ],
        out_specs=pl.BlockSpec((TM, H), lambda t, e: (t, 0)),
        out_shape=jax.ShapeDtypeStruct((T, H), jnp.float32),
        scratch_shapes=[pltpu.VMEM((TM, E), jnp.float32)],
    )(x, gate_weight, Wg, Wu, Wd)

    out = pl.pallas_call(
        _shared_kernel,
        grid=(T // TM, N_SHARED),
        in_specs=[
            pl.BlockSpec((TM, H), lambda t, c: (t, 0)),
            pl.BlockSpec((D_FF, H), lambda t, c: (c, 0)),
            pl.BlockSpec((D_FF, H), lambda t, c: (c, 0)),
            pl.BlockSpec((H, D_FF), lambda t, c: (0, c)),
            pl.BlockSpec((TM, H), lambda t, c: (t, 0)),
        ],
        out_specs=pl.BlockSpec((TM, H), lambda t, c: (t, 0)),
        out_shape=jax.ShapeDtypeStruct((T, H), jnp.float32),
        input_output_aliases={4: 0},
    )(x, Wsg, Wsu, Wsd, routed)

    return out.reshape(bsz, seq_len, h)
